# baseline (speedup 1.0000x reference)
import jax
import jax.numpy as jnp
from jax.experimental import pallas as pl
from jax.experimental.pallas import tpu as pltpu

KEY_DIM = 4096
PFC_DIM = 1024
N_PATCHES = 4
N_SLOTS = 1024
EPS = 8
D_MEM = PFC_DIM + N_PATCHES * 3
IN_DIM = KEY_DIM + PFC_DIM
H_DIM = KEY_DIM * 2

BM0 = 256
BM2 = 256
BP = 256

_NEG = float('-inf')


def _matvec_bias_kernel(w_ref, x_ref, b_ref, o_ref, *, relu):
    acc = jax.lax.dot_general(
        w_ref[...], x_ref[...], (((1,), (0,)), ((), ())),
        preferred_element_type=jnp.float32) + b_ref[...]
    if relu:
        acc = jnp.maximum(acc, 0.0)
    o_ref[...] = acc


def _stage3_kernel(p_ref, k_ref, slot_ref, sim_ref, bv_ref, bi_ref):
    i = pl.program_id(0)
    n = pl.num_programs(0)

    @pl.when(i == 0)
    def _init():
        bv_ref[...] = jnp.full((1, 1), _NEG, jnp.float32)
        bi_ref[...] = jnp.zeros((1, 1), jnp.int32)

    k = k_ref[...]
    knorm = jnp.clip(jnp.sqrt(jnp.sum(k * k)), 1e-12, None)
    p = p_ref[...]
    raw = jax.lax.dot_general(
        p, k, (((1,), (0,)), ((), ())),
        preferred_element_type=jnp.float32)
    pn = jnp.clip(jnp.sqrt(jnp.sum(p * p, axis=1, keepdims=True)), 1e-12, None)
    sims = raw / (pn * knorm)

    mx = jnp.max(sims, axis=0, keepdims=True)
    idx = jax.lax.broadcasted_iota(jnp.int32, (BP, 1), 0) + i * BP
    bidx = jnp.min(jnp.where(sims == mx, idx, jnp.int32(2**30)),
                   axis=0, keepdims=True)
    better = mx > bv_ref[...]
    bi_ref[...] = jnp.where(better, bidx, bi_ref[...])
    bv_ref[...] = jnp.where(better, mx, bv_ref[...])

    @pl.when(i == n - 1)
    def _fin():
        slot_ref[...] = bi_ref[...]
        sim_ref[...] = bv_ref[...]


def _stage4_kernel(slot_pref, ep_ref, td_ref, cnt_ref, sim_ref, pfc_row_ref,
                   pfc_col_ref, tde_ref, wg1_ref, bg1_ref, wg2_ref, bg2_ref,
                   wr_ref, br_ref, wn_ref, bn_ref,
                   newpfc_ref, alpha_ref, onehot_ref, nm_ref):
    eps = ep_ref[0]
    stored = eps[:, :PFC_DIM]
    pfc_row = pfc_row_ref[...]
    pnorm = jnp.clip(jnp.sqrt(jnp.sum(pfc_row * pfc_row)), 1e-12, None)
    pn = pfc_row / pnorm
    snorm = jnp.clip(jnp.sqrt(jnp.sum(stored * stored, axis=1, keepdims=True)),
                     1e-12, None)
    sims_e = jnp.sum(stored * pn, axis=1, keepdims=True) / snorm

    td = td_ref[0]
    rel = sims_e * jnp.clip(jnp.abs(td), 1e-6, None)
    n_eps = jnp.minimum(cnt_ref[0], EPS)
    idx8 = jax.lax.broadcasted_iota(jnp.int32, (EPS, 1), 0)
    rel = jnp.where(idx8 < n_eps, rel, _NEG)
    mx = jnp.max(rel, axis=0, keepdims=True)
    bidx = jnp.min(jnp.where(rel == mx, idx8, jnp.int32(2**30)),
                   axis=0, keepdims=True)
    oh8 = (idx8 == bidx).astype(jnp.float32)
    ep_content = jnp.sum(eps * oh8, axis=0, keepdims=True)
    ep_td = jnp.sum(td * oh8, axis=0, keepdims=True)

    wg1 = wg1_ref[...]
    x0 = sim_ref[...]
    x1 = jnp.abs(tde_ref[...])
    g = jnp.tanh(wg1[:, 0:1] * x0 + wg1[:, 1:2] * x1 + wg1[:, 2:3] * ep_td
                 + bg1_ref[...])
    alpha = jnp.tanh(jnp.sum(wg2_ref[...] * g, axis=0, keepdims=True)
                     + bg2_ref[...])

    delta = jnp.sum(wr_ref[...] * ep_content, axis=1, keepdims=True) \
        + br_ref[...]
    newpfc_ref[...] = pfc_col_ref[...] + alpha * delta
    alpha_ref[...] = alpha

    slot = slot_pref[0]
    ii = jax.lax.broadcasted_iota(jnp.int32, (N_SLOTS, 1), 0)
    onehot_ref[...] = (ii == slot).astype(jnp.float32)

    nm = jnp.sum(wn_ref[...] * ep_content, axis=1, keepdims=True) + bn_ref[...]
    rows = jax.lax.broadcasted_iota(jnp.int32, (3 * N_PATCHES, 1), 0)
    hi = jnp.where(rows < 2 * N_PATCHES, 1.0, 0.5)
    nm_ref[...] = jnp.clip(nm, 0.1, hi)


def _matvec_call(w, x, b, bm, relu):
    m, k = w.shape
    import functools
    return pl.pallas_call(
        functools.partial(_matvec_bias_kernel, relu=relu),
        grid=(m // bm,),
        in_specs=[
            pl.BlockSpec((bm, k), lambda i: (i, 0)),
            pl.BlockSpec((k, 1), lambda i: (0, 0)),
            pl.BlockSpec((bm, 1), lambda i: (i, 0)),
        ],
        out_specs=pl.BlockSpec((bm, 1), lambda i: (i, 0)),
        out_shape=jax.ShapeDtypeStruct((m, 1), jnp.float32),
    )(w, x, b)


def kernel(activation_summary, pfc_state, current_td_error, prototypes,
           log_temperature, W0, b0, W2, b2, episodes, ep_td_errors, ep_count,
           Wg1, bg1, Wg2, bg2, Wr, br, Wn, bn):
    f32 = jnp.float32
    combined = jnp.concatenate(
        [activation_summary.reshape(KEY_DIM, 1),
         pfc_state.reshape(PFC_DIM, 1)], axis=0)

    h_col = _matvec_call(W0, combined, b0.reshape(H_DIM, 1), BM0, relu=True)
    key_col = _matvec_call(W2, h_col, b2.reshape(KEY_DIM, 1),
                           BM2, relu=False)

    slot11, sim11 = pl.pallas_call(
        _stage3_kernel,
        grid=(N_SLOTS // BP,),
        in_specs=[
            pl.BlockSpec((BP, KEY_DIM), lambda i: (i, 0)),
            pl.BlockSpec((KEY_DIM, 1), lambda i: (0, 0)),
        ],
        out_specs=[
            pl.BlockSpec((1, 1), lambda i: (0, 0)),
            pl.BlockSpec((1, 1), lambda i: (0, 0)),
        ],
        out_shape=[
            jax.ShapeDtypeStruct((1, 1), jnp.int32),
            jax.ShapeDtypeStruct((1, 1), jnp.float32),
        ],
        scratch_shapes=[
            pltpu.VMEM((1, 1), jnp.float32),
            pltpu.VMEM((1, 1), jnp.int32),
        ],
    )(prototypes, key_col)

    slot1 = slot11.reshape((1,))

    grid_spec = pltpu.PrefetchScalarGridSpec(
        num_scalar_prefetch=1,
        grid=(1,),
        in_specs=[
            pl.BlockSpec((1, EPS, D_MEM), lambda i, s: (s[0], 0, 0)),
            pl.BlockSpec((1, EPS, 1), lambda i, s: (s[0], 0, 0)),
            pl.BlockSpec((1, 1, 1), lambda i, s: (s[0], 0, 0)),
            pl.BlockSpec((1, 1), lambda i, s: (0, 0)),
            pl.BlockSpec((1, PFC_DIM), lambda i, s: (0, 0)),
            pl.BlockSpec((PFC_DIM, 1), lambda i, s: (0, 0)),
            pl.BlockSpec((1, 1), lambda i, s: (0, 0)),
            pl.BlockSpec((16, 3), lambda i, s: (0, 0)),
            pl.BlockSpec((16, 1), lambda i, s: (0, 0)),
            pl.BlockSpec((16, 1), lambda i, s: (0, 0)),
            pl.BlockSpec((1, 1), lambda i, s: (0, 0)),
            pl.BlockSpec((PFC_DIM, D_MEM), lambda i, s: (0, 0)),
            pl.BlockSpec((PFC_DIM, 1), lambda i, s: (0, 0)),
            pl.BlockSpec((3 * N_PATCHES, D_MEM), lambda i, s: (0, 0)),
            pl.BlockSpec((3 * N_PATCHES, 1), lambda i, s: (0, 0)),
        ],
        out_specs=[
            pl.BlockSpec((PFC_DIM, 1), lambda i, s: (0, 0)),
            pl.BlockSpec((1, 1), lambda i, s: (0, 0)),
            pl.BlockSpec((N_SLOTS, 1), lambda i, s: (0, 0)),
            pl.BlockSpec((3 * N_PATCHES, 1), lambda i, s: (0, 0)),
        ],
    )

    newpfc, alpha11, onehot, nm = pl.pallas_call(
        _stage4_kernel,
        grid_spec=grid_spec,
        out_shape=[
            jax.ShapeDtypeStruct((PFC_DIM, 1), f32),
            jax.ShapeDtypeStruct((1, 1), f32),
            jax.ShapeDtypeStruct((N_SLOTS, 1), f32),
            jax.ShapeDtypeStruct((3 * N_PATCHES, 1), f32),
        ],
    )(slot1, episodes, ep_td_errors.reshape(N_SLOTS, EPS, 1),
      ep_count.reshape(N_SLOTS, 1, 1), sim11, pfc_state,
      pfc_state.reshape(PFC_DIM, 1), current_td_error.reshape(1, 1),
      Wg1, bg1.reshape(16, 1), Wg2.reshape(16, 1), bg2.reshape(1, 1),
      Wr, br.reshape(PFC_DIM, 1), Wn, bn.reshape(3 * N_PATCHES, 1))

    new_pfc = newpfc.reshape(1, PFC_DIM)
    alpha = alpha11.reshape(())
    one_hot_st = onehot.reshape(N_SLOTS)
    nmflat = nm.reshape(3 * N_PATCHES)
    eta = nmflat[0:N_PATCHES]
    decay = nmflat[N_PATCHES:2 * N_PATCHES]
    expl = nmflat[2 * N_PATCHES:]
    return (new_pfc, alpha, one_hot_st, eta, decay, expl)


# speedup vs baseline: 1.0962x; 1.0962x over previous
import jax
import jax.numpy as jnp
from jax.experimental import pallas as pl
from jax.experimental.pallas import tpu as pltpu

KEY_DIM = 4096
PFC_DIM = 1024
N_PATCHES = 4
N_SLOTS = 1024
EPS = 8
D_MEM = PFC_DIM + N_PATCHES * 3
IN_DIM = KEY_DIM + PFC_DIM
H_DIM = KEY_DIM * 2

BM = 256
NB_A = H_DIM // BM
NB_B = KEY_DIM // BM
NB_C = N_SLOTS // BM

_NEG = float('-inf')


def _fused_kernel(comb_ref, b0_ref, b2_ref, pfc_row_ref, pfc_col_ref,
                  tde_ref, wg1_ref, bg1_ref, wg2_ref, bg2_ref, br_ref,
                  wn_ref, bn_ref,
                  w0_hbm, w2_hbm, proto_hbm, ep_hbm, td_hbm, cnt_hbm, wr_hbm,
                  newpfc_ref, alpha_ref, onehot_ref, nm_ref,
                  wbuf, wrbuf, epbuf, tdbuf, cntbuf, h_row, key_row,
                  best_val, best_idx, sems, wrsem, epsem):

    def w0_copy(i, p):
        return pltpu.make_async_copy(
            w0_hbm.at[pl.ds(i * BM, BM), :], wbuf.at[p, :, pl.ds(0, IN_DIM)],
            sems.at[p])

    def w2_copy(i, p):
        return pltpu.make_async_copy(
            w2_hbm.at[pl.ds(i * BM, BM), :], wbuf.at[p], sems.at[p])

    def proto_copy(i, p):
        return pltpu.make_async_copy(
            proto_hbm.at[pl.ds(i * BM, BM), :],
            wbuf.at[p, :, pl.ds(0, KEY_DIM)], sems.at[p])

    w0_copy(0, 0).start()
    pltpu.make_async_copy(wr_hbm, wrbuf, wrsem).start()

    comb = comb_ref[...]

    def body_a(i, _):
        p = jax.lax.rem(i, 2)
        pnext = jax.lax.rem(i + 1, 2)

        @pl.when(i + 1 < NB_A)
        def _():
            w0_copy(i + 1, pnext).start()

        @pl.when(i + 1 == NB_A)
        def _():
            w2_copy(0, pnext).start()

        w0_copy(i, p).wait()
        w = wbuf[p, :, pl.ds(0, IN_DIM)]
        col = jnp.sum(w * comb, axis=1, keepdims=True) + b0_ref[pl.ds(i * BM, BM), :]
        col = jnp.maximum(col, 0.0)
        h_row[:, pl.ds(i * BM, BM)] = col.T
        return 0

    jax.lax.fori_loop(0, NB_A, body_a, 0)

    def body_b(i, _):
        p = jax.lax.rem(i, 2)
        pnext = jax.lax.rem(i + 1, 2)

        @pl.when(i + 1 < NB_B)
        def _():
            w2_copy(i + 1, pnext).start()

        @pl.when(i + 1 == NB_B)
        def _():
            proto_copy(0, pnext).start()

        w2_copy(i, p).wait()
        w = wbuf[p]
        col = jnp.sum(w * h_row[...], axis=1, keepdims=True) \
            + b2_ref[pl.ds(i * BM, BM), :]
        key_row[:, pl.ds(i * BM, BM)] = col.T
        return 0

    jax.lax.fori_loop(0, NB_B, body_b, 0)

    best_val[0] = _NEG
    best_idx[0] = 0
    key = key_row[...]
    knorm = jnp.clip(jnp.sqrt(jnp.sum(key * key)), 1e-12, None)

    def body_c(i, _):
        p = jax.lax.rem(i, 2)
        pnext = jax.lax.rem(i + 1, 2)

        @pl.when(i + 1 < NB_C)
        def _():
            proto_copy(i + 1, pnext).start()

        proto_copy(i, p).wait()
        w = wbuf[p, :, pl.ds(0, KEY_DIM)]
        raw = jnp.sum(w * key, axis=1, keepdims=True)
        pn = jnp.clip(jnp.sqrt(jnp.sum(w * w, axis=1, keepdims=True)),
                      1e-12, None)
        sims = raw / (pn * knorm)
        mx = jnp.max(sims)
        idx = jax.lax.broadcasted_iota(jnp.int32, (BM, 1), 0) + i * BM
        bidx = jnp.min(jnp.where(sims == mx, idx, jnp.int32(2**30)))

        @pl.when(mx > best_val[0])
        def _():
            best_val[0] = mx
            best_idx[0] = bidx

        return 0

    jax.lax.fori_loop(0, NB_C, body_c, 0)

    slot = best_idx[0]
    pltpu.make_async_copy(ep_hbm.at[pl.ds(slot, 1)], epbuf, epsem).start()
    pltpu.make_async_copy(ep_hbm.at[pl.ds(slot, 1)], epbuf, epsem).wait()
    pltpu.make_async_copy(td_hbm.at[pl.ds(slot, 1)], tdbuf, epsem).start()
    pltpu.make_async_copy(td_hbm.at[pl.ds(slot, 1)], tdbuf, epsem).wait()
    pltpu.make_async_copy(cnt_hbm.at[pl.ds(slot, 1)], cntbuf, epsem).start()
    pltpu.make_async_copy(cnt_hbm.at[pl.ds(slot, 1)], cntbuf, epsem).wait()
    pltpu.make_async_copy(wr_hbm, wrbuf, wrsem).wait()

    eps = epbuf[0]
    stored = eps[:, :PFC_DIM]
    pfc_row = pfc_row_ref[...]
    pnorm = jnp.clip(jnp.sqrt(jnp.sum(pfc_row * pfc_row)), 1e-12, None)
    pn = pfc_row / pnorm
    snorm = jnp.clip(jnp.sqrt(jnp.sum(stored * stored, axis=1, keepdims=True)),
                     1e-12, None)
    sims_e = jnp.sum(stored * pn, axis=1, keepdims=True) / snorm

    td = tdbuf[0]
    rel = sims_e * jnp.clip(jnp.abs(td), 1e-6, None)
    n_eps = jnp.minimum(cntbuf[0], EPS)
    idx8 = jax.lax.broadcasted_iota(jnp.int32, (EPS, 1), 0)
    rel = jnp.where(idx8 < n_eps, rel, _NEG)
    mx = jnp.max(rel, axis=0, keepdims=True)
    bidx = jnp.min(jnp.where(rel == mx, idx8, jnp.int32(2**30)),
                   axis=0, keepdims=True)
    oh8 = (idx8 == bidx).astype(jnp.float32)
    ep_content = jnp.sum(eps * oh8, axis=0, keepdims=True)
    ep_td = jnp.sum(td * oh8, axis=0, keepdims=True)

    wg1 = wg1_ref[...]
    x1 = jnp.abs(tde_ref[...])
    g = jnp.tanh(wg1[:, 0:1] * best_val[0] + wg1[:, 1:2] * x1
                 + wg1[:, 2:3] * ep_td + bg1_ref[...])
    alpha = jnp.tanh(jnp.sum(wg2_ref[...] * g, axis=0, keepdims=True)
                     + bg2_ref[...])

    delta = jnp.sum(wrbuf[...] * ep_content, axis=1, keepdims=True) \
        + br_ref[...]
    newpfc_ref[...] = pfc_col_ref[...] + alpha * delta
    alpha_ref[...] = alpha

    ii = jax.lax.broadcasted_iota(jnp.int32, (N_SLOTS, 1), 0)
    onehot_ref[...] = (ii == slot).astype(jnp.float32)

    nm = jnp.sum(wn_ref[...] * ep_content, axis=1, keepdims=True) + bn_ref[...]
    rows = jax.lax.broadcasted_iota(jnp.int32, (3 * N_PATCHES, 1), 0)
    hi = jnp.where(rows < 2 * N_PATCHES, 1.0, 0.5)
    nm_ref[...] = jnp.clip(nm, 0.1, hi)


def kernel(activation_summary, pfc_state, current_td_error, prototypes,
           log_temperature, W0, b0, W2, b2, episodes, ep_td_errors, ep_count,
           Wg1, bg1, Wg2, bg2, Wr, br, Wn, bn):
    f32 = jnp.float32
    combined = jnp.concatenate(
        [activation_summary.reshape(1, KEY_DIM), pfc_state], axis=1)

    vm = pl.BlockSpec(memory_space=pl.ANY)
    newpfc, alpha11, onehot, nm = pl.pallas_call(
        _fused_kernel,
        in_specs=[pl.BlockSpec(memory_space=pltpu.VMEM)] * 13
        + [vm] * 7,
        out_specs=[pl.BlockSpec(memory_space=pltpu.VMEM)] * 4,
        out_shape=[
            jax.ShapeDtypeStruct((PFC_DIM, 1), f32),
            jax.ShapeDtypeStruct((1, 1), f32),
            jax.ShapeDtypeStruct((N_SLOTS, 1), f32),
            jax.ShapeDtypeStruct((3 * N_PATCHES, 1), f32),
        ],
        scratch_shapes=[
            pltpu.VMEM((2, BM, H_DIM), f32),
            pltpu.VMEM((PFC_DIM, D_MEM), f32),
            pltpu.VMEM((1, EPS, D_MEM), f32),
            pltpu.VMEM((1, EPS, 1), f32),
            pltpu.VMEM((1, 1, 1), jnp.int32),
            pltpu.VMEM((1, H_DIM), f32),
            pltpu.VMEM((1, KEY_DIM), f32),
            pltpu.SMEM((1,), f32),
            pltpu.SMEM((1,), jnp.int32),
            pltpu.SemaphoreType.DMA((2,)),
            pltpu.SemaphoreType.DMA,
            pltpu.SemaphoreType.DMA,
        ],
    )(combined, b0.reshape(H_DIM, 1), b2.reshape(KEY_DIM, 1),
      pfc_state, pfc_state.reshape(PFC_DIM, 1),
      current_td_error.reshape(1, 1), Wg1, bg1.reshape(16, 1),
      Wg2.reshape(16, 1), bg2.reshape(1, 1), br.reshape(PFC_DIM, 1),
      Wn, bn.reshape(3 * N_PATCHES, 1),
      W0, W2, prototypes, episodes, ep_td_errors.reshape(N_SLOTS, EPS, 1),
      ep_count.reshape(N_SLOTS, 1, 1), Wr)

    new_pfc = newpfc.reshape(1, PFC_DIM)
    alpha = alpha11.reshape(())
    one_hot_st = onehot.reshape(N_SLOTS)
    nmflat = nm.reshape(3 * N_PATCHES)
    eta = nmflat[0:N_PATCHES]
    decay = nmflat[N_PATCHES:2 * N_PATCHES]
    expl = nmflat[2 * N_PATCHES:]
    return (new_pfc, alpha, one_hot_st, eta, decay, expl)


# speedup vs baseline: 1.0988x; 1.0023x over previous
import jax
import jax.numpy as jnp
from jax.experimental import pallas as pl
from jax.experimental.pallas import tpu as pltpu

KEY_DIM = 4096
PFC_DIM = 1024
N_PATCHES = 4
N_SLOTS = 1024
EPS = 8
D_MEM = PFC_DIM + N_PATCHES * 3
IN_DIM = KEY_DIM + PFC_DIM
H_DIM = KEY_DIM * 2

BM = 256
NB_A = H_DIM // BM
NB_B = KEY_DIM // BM
NB_C = N_SLOTS // BM
NS = 4
SR = BM // NS

_NEG = float('-inf')


def _fused_kernel(comb_ref, b0_ref, b2_ref, pfc_row_ref, pfc_col_ref,
                  tde_ref, wg1_ref, bg1_ref, wg2_ref, bg2_ref, br_ref,
                  wn_ref, bn_ref,
                  w0_hbm, w2_hbm, proto_hbm, ep_hbm, td_hbm, cnt_hbm, wr_hbm,
                  newpfc_ref, alpha_ref, onehot_ref, nm_ref,
                  wbuf, wrbuf, epbuf, tdbuf, cntbuf, h_row, key_row,
                  best_val, best_idx, sems, wrsem, epsem):

    def _striped(hbm, width, i, p, s):
        return pltpu.make_async_copy(
            hbm.at[pl.ds(i * BM + s * SR, SR), :],
            wbuf.at[p, pl.ds(s * SR, SR), pl.ds(0, width)],
            sems.at[p, s])

    class _blk:
        def __init__(self, hbm, width, i, p):
            self.copies = [_striped(hbm, width, i, p, s) for s in range(NS)]

        def start(self):
            for c in self.copies:
                c.start()

        def wait(self):
            for c in self.copies:
                c.wait()

    def w0_copy(i, p):
        return _blk(w0_hbm, IN_DIM, i, p)

    def w2_copy(i, p):
        return _blk(w2_hbm, H_DIM, i, p)

    def proto_copy(i, p):
        return _blk(proto_hbm, KEY_DIM, i, p)

    w0_copy(0, 0).start()
    pltpu.make_async_copy(wr_hbm, wrbuf, wrsem).start()

    comb = comb_ref[...]

    def body_a(i, _):
        p = jax.lax.rem(i, 2)
        pnext = jax.lax.rem(i + 1, 2)

        @pl.when(i + 1 < NB_A)
        def _():
            w0_copy(i + 1, pnext).start()

        @pl.when(i + 1 == NB_A)
        def _():
            w2_copy(0, pnext).start()

        w0_copy(i, p).wait()
        w = wbuf[p, :, pl.ds(0, IN_DIM)]
        col = jnp.sum(w * comb, axis=1, keepdims=True) + b0_ref[pl.ds(i * BM, BM), :]
        col = jnp.maximum(col, 0.0)
        h_row[:, pl.ds(i * BM, BM)] = col.T
        return 0

    jax.lax.fori_loop(0, NB_A, body_a, 0)

    def body_b(i, _):
        p = jax.lax.rem(i, 2)
        pnext = jax.lax.rem(i + 1, 2)

        @pl.when(i + 1 < NB_B)
        def _():
            w2_copy(i + 1, pnext).start()

        @pl.when(i + 1 == NB_B)
        def _():
            proto_copy(0, pnext).start()

        w2_copy(i, p).wait()
        w = wbuf[p]
        col = jnp.sum(w * h_row[...], axis=1, keepdims=True) \
            + b2_ref[pl.ds(i * BM, BM), :]
        key_row[:, pl.ds(i * BM, BM)] = col.T
        return 0

    jax.lax.fori_loop(0, NB_B, body_b, 0)

    best_val[0] = _NEG
    best_idx[0] = 0
    key = key_row[...]
    knorm = jnp.clip(jnp.sqrt(jnp.sum(key * key)), 1e-12, None)

    def body_c(i, _):
        p = jax.lax.rem(i, 2)
        pnext = jax.lax.rem(i + 1, 2)

        @pl.when(i + 1 < NB_C)
        def _():
            proto_copy(i + 1, pnext).start()

        proto_copy(i, p).wait()
        w = wbuf[p, :, pl.ds(0, KEY_DIM)]
        raw = jnp.sum(w * key, axis=1, keepdims=True)
        pn = jnp.clip(jnp.sqrt(jnp.sum(w * w, axis=1, keepdims=True)),
                      1e-12, None)
        sims = raw / (pn * knorm)
        mx = jnp.max(sims)
        idx = jax.lax.broadcasted_iota(jnp.int32, (BM, 1), 0) + i * BM
        bidx = jnp.min(jnp.where(sims == mx, idx, jnp.int32(2**30)))

        @pl.when(mx > best_val[0])
        def _():
            best_val[0] = mx
            best_idx[0] = bidx

        return 0

    jax.lax.fori_loop(0, NB_C, body_c, 0)

    slot = best_idx[0]
    pltpu.make_async_copy(ep_hbm.at[pl.ds(slot, 1)], epbuf, epsem).start()
    pltpu.make_async_copy(ep_hbm.at[pl.ds(slot, 1)], epbuf, epsem).wait()
    pltpu.make_async_copy(td_hbm.at[pl.ds(slot, 1)], tdbuf, epsem).start()
    pltpu.make_async_copy(td_hbm.at[pl.ds(slot, 1)], tdbuf, epsem).wait()
    pltpu.make_async_copy(cnt_hbm.at[pl.ds(slot, 1)], cntbuf, epsem).start()
    pltpu.make_async_copy(cnt_hbm.at[pl.ds(slot, 1)], cntbuf, epsem).wait()
    pltpu.make_async_copy(wr_hbm, wrbuf, wrsem).wait()

    eps = epbuf[0]
    stored = eps[:, :PFC_DIM]
    pfc_row = pfc_row_ref[...]
    pnorm = jnp.clip(jnp.sqrt(jnp.sum(pfc_row * pfc_row)), 1e-12, None)
    pn = pfc_row / pnorm
    snorm = jnp.clip(jnp.sqrt(jnp.sum(stored * stored, axis=1, keepdims=True)),
                     1e-12, None)
    sims_e = jnp.sum(stored * pn, axis=1, keepdims=True) / snorm

    td = tdbuf[0]
    rel = sims_e * jnp.clip(jnp.abs(td), 1e-6, None)
    n_eps = jnp.minimum(cntbuf[0], EPS)
    idx8 = jax.lax.broadcasted_iota(jnp.int32, (EPS, 1), 0)
    rel = jnp.where(idx8 < n_eps, rel, _NEG)
    mx = jnp.max(rel, axis=0, keepdims=True)
    bidx = jnp.min(jnp.where(rel == mx, idx8, jnp.int32(2**30)),
                   axis=0, keepdims=True)
    oh8 = (idx8 == bidx).astype(jnp.float32)
    ep_content = jnp.sum(eps * oh8, axis=0, keepdims=True)
    ep_td = jnp.sum(td * oh8, axis=0, keepdims=True)

    wg1 = wg1_ref[...]
    x1 = jnp.abs(tde_ref[...])
    g = jnp.tanh(wg1[:, 0:1] * best_val[0] + wg1[:, 1:2] * x1
                 + wg1[:, 2:3] * ep_td + bg1_ref[...])
    alpha = jnp.tanh(jnp.sum(wg2_ref[...] * g, axis=0, keepdims=True)
                     + bg2_ref[...])

    delta = jnp.sum(wrbuf[...] * ep_content, axis=1, keepdims=True) \
        + br_ref[...]
    newpfc_ref[...] = pfc_col_ref[...] + alpha * delta
    alpha_ref[...] = alpha

    ii = jax.lax.broadcasted_iota(jnp.int32, (N_SLOTS, 1), 0)
    onehot_ref[...] = (ii == slot).astype(jnp.float32)

    nm = jnp.sum(wn_ref[...] * ep_content, axis=1, keepdims=True) + bn_ref[...]
    rows = jax.lax.broadcasted_iota(jnp.int32, (3 * N_PATCHES, 1), 0)
    hi = jnp.where(rows < 2 * N_PATCHES, 1.0, 0.5)
    nm_ref[...] = jnp.clip(nm, 0.1, hi)


def kernel(activation_summary, pfc_state, current_td_error, prototypes,
           log_temperature, W0, b0, W2, b2, episodes, ep_td_errors, ep_count,
           Wg1, bg1, Wg2, bg2, Wr, br, Wn, bn):
    f32 = jnp.float32
    combined = jnp.concatenate(
        [activation_summary.reshape(1, KEY_DIM), pfc_state], axis=1)

    vm = pl.BlockSpec(memory_space=pl.ANY)
    newpfc, alpha11, onehot, nm = pl.pallas_call(
        _fused_kernel,
        in_specs=[pl.BlockSpec(memory_space=pltpu.VMEM)] * 13
        + [vm] * 7,
        out_specs=[pl.BlockSpec(memory_space=pltpu.VMEM)] * 4,
        out_shape=[
            jax.ShapeDtypeStruct((PFC_DIM, 1), f32),
            jax.ShapeDtypeStruct((1, 1), f32),
            jax.ShapeDtypeStruct((N_SLOTS, 1), f32),
            jax.ShapeDtypeStruct((3 * N_PATCHES, 1), f32),
        ],
        scratch_shapes=[
            pltpu.VMEM((2, BM, H_DIM), f32),
            pltpu.VMEM((PFC_DIM, D_MEM), f32),
            pltpu.VMEM((1, EPS, D_MEM), f32),
            pltpu.VMEM((1, EPS, 1), f32),
            pltpu.VMEM((1, 1, 1), jnp.int32),
            pltpu.VMEM((1, H_DIM), f32),
            pltpu.VMEM((1, KEY_DIM), f32),
            pltpu.SMEM((1,), f32),
            pltpu.SMEM((1,), jnp.int32),
            pltpu.SemaphoreType.DMA((2, NS)),
            pltpu.SemaphoreType.DMA,
            pltpu.SemaphoreType.DMA,
        ],
    )(combined, b0.reshape(H_DIM, 1), b2.reshape(KEY_DIM, 1),
      pfc_state, pfc_state.reshape(PFC_DIM, 1),
      current_td_error.reshape(1, 1), Wg1, bg1.reshape(16, 1),
      Wg2.reshape(16, 1), bg2.reshape(1, 1), br.reshape(PFC_DIM, 1),
      Wn, bn.reshape(3 * N_PATCHES, 1),
      W0, W2, prototypes, episodes, ep_td_errors.reshape(N_SLOTS, EPS, 1),
      ep_count.reshape(N_SLOTS, 1, 1), Wr)

    new_pfc = newpfc.reshape(1, PFC_DIM)
    alpha = alpha11.reshape(())
    one_hot_st = onehot.reshape(N_SLOTS)
    nmflat = nm.reshape(3 * N_PATCHES)
    eta = nmflat[0:N_PATCHES]
    decay = nmflat[N_PATCHES:2 * N_PATCHES]
    expl = nmflat[2 * N_PATCHES:]
    return (new_pfc, alpha, one_hot_st, eta, decay, expl)


# speedup vs baseline: 1.1887x; 1.0819x over previous
import jax
import jax.numpy as jnp
from jax.experimental import pallas as pl
from jax.experimental.pallas import tpu as pltpu

KEY_DIM = 4096
PFC_DIM = 1024
N_PATCHES = 4
N_SLOTS = 1024
EPS = 8
D_MEM = PFC_DIM + N_PATCHES * 3
IN_DIM = KEY_DIM + PFC_DIM
H_DIM = KEY_DIM * 2

BM = 256
NB_A = H_DIM // BM
NB_B = KEY_DIM // BM
NB_C = N_SLOTS // BM
NB = NB_A + NB_B + NB_C
NRING = 5

_NEG = float('-inf')


def _fused_kernel(comb_ref, b0_ref, b2_ref, pfc_row_ref, pfc_col_ref,
                  tde_ref, wg1_ref, bg1_ref, wg2_ref, bg2_ref, br_ref,
                  wn_ref, bn_ref,
                  w0_hbm, w2_hbm, proto_hbm, ep_hbm, td_hbm, cnt_hbm, wr_hbm,
                  newpfc_ref, alpha_ref, onehot_ref, nm_ref,
                  wbuf, wrbuf, epbuf, tdbuf, cntbuf, h_row, key_row,
                  best_val, best_idx, sems, wrsem):

    def w0_copy(i, p):
        return pltpu.make_async_copy(
            w0_hbm.at[pl.ds(i * BM, BM), :], wbuf.at[p, :, pl.ds(0, IN_DIM)],
            sems.at[p])

    def w2_copy(i, p):
        return pltpu.make_async_copy(
            w2_hbm.at[pl.ds(i * BM, BM), :], wbuf.at[p], sems.at[p])

    def proto_copy(i, p):
        return pltpu.make_async_copy(
            proto_hbm.at[pl.ds(i * BM, BM), :],
            wbuf.at[p, :, pl.ds(0, KEY_DIM)], sems.at[p])

    def start_block(g):
        p = jax.lax.rem(g, NRING)

        @pl.when(g < NB_A)
        def _():
            w0_copy(g, p).start()

        @pl.when(jnp.logical_and(g >= NB_A, g < NB_A + NB_B))
        def _():
            w2_copy(g - NB_A, p).start()

        @pl.when(jnp.logical_and(g >= NB_A + NB_B, g < NB))
        def _():
            proto_copy(g - NB_A - NB_B, p).start()

    pltpu.make_async_copy(wr_hbm, wrbuf, wrsem).start()
    for g in range(NRING - 1):
        w0_copy(g, g % NRING).start()

    comb = comb_ref[...]

    def body_a(i, _):
        p = jax.lax.rem(i, NRING)
        start_block(i + NRING - 1)
        w0_copy(i, p).wait()
        w = wbuf[p, :, pl.ds(0, IN_DIM)]
        col = jnp.sum(w * comb, axis=1, keepdims=True)
        h_row[:, pl.ds(i * BM, BM)] = jnp.maximum(
            col.T + b0_ref[:, pl.ds(i * BM, BM)], 0.0)
        return 0

    jax.lax.fori_loop(0, NB_A, body_a, 0)

    def body_b(i, _):
        g = NB_A + i
        p = jax.lax.rem(g, NRING)
        start_block(g + NRING - 1)
        w2_copy(i, p).wait()
        w = wbuf[p]
        col = jnp.sum(w * h_row[...], axis=1, keepdims=True)
        key_row[:, pl.ds(i * BM, BM)] = col.T + b2_ref[:, pl.ds(i * BM, BM)]
        return 0

    jax.lax.fori_loop(0, NB_B, body_b, 0)

    best_val[0] = _NEG
    best_idx[0] = 0
    key = key_row[...]
    knorm = jnp.clip(jnp.sqrt(jnp.sum(key * key)), 1e-12, None)

    def body_c(i, _):
        g = NB_A + NB_B + i
        p = jax.lax.rem(g, NRING)
        proto_copy(i, p).wait()
        w = wbuf[p, :, pl.ds(0, KEY_DIM)]
        raw = jnp.sum(w * key, axis=1, keepdims=True)
        pn = jnp.clip(jnp.sqrt(jnp.sum(w * w, axis=1, keepdims=True)),
                      1e-12, None)
        sims = raw / (pn * knorm)
        mx = jnp.max(sims)
        idx = jax.lax.broadcasted_iota(jnp.int32, (BM, 1), 0) + i * BM
        bidx = jnp.min(jnp.where(sims == mx, idx, jnp.int32(2**30)))

        @pl.when(mx > best_val[0])
        def _():
            best_val[0] = mx
            best_idx[0] = bidx

        return 0

    jax.lax.fori_loop(0, NB_C, body_c, 0)

    slot = best_idx[0]
    ep_cp = pltpu.make_async_copy(ep_hbm.at[pl.ds(slot, 1)], epbuf, sems.at[0])
    td_cp = pltpu.make_async_copy(td_hbm.at[pl.ds(slot, 1)], tdbuf, sems.at[1])
    cnt_cp = pltpu.make_async_copy(cnt_hbm.at[pl.ds(slot, 1)], cntbuf,
                                   sems.at[2])
    ep_cp.start()
    td_cp.start()
    cnt_cp.start()
    ep_cp.wait()
    td_cp.wait()
    cnt_cp.wait()
    pltpu.make_async_copy(wr_hbm, wrbuf, wrsem).wait()

    eps = epbuf[0]
    stored = eps[:, :PFC_DIM]
    pfc_row = pfc_row_ref[...]
    pnorm = jnp.clip(jnp.sqrt(jnp.sum(pfc_row * pfc_row)), 1e-12, None)
    pn = pfc_row / pnorm
    snorm = jnp.clip(jnp.sqrt(jnp.sum(stored * stored, axis=1, keepdims=True)),
                     1e-12, None)
    sims_e = jnp.sum(stored * pn, axis=1, keepdims=True) / snorm

    td = tdbuf[0]
    rel = sims_e * jnp.clip(jnp.abs(td), 1e-6, None)
    n_eps = jnp.minimum(cntbuf[0], EPS)
    idx8 = jax.lax.broadcasted_iota(jnp.int32, (EPS, 1), 0)
    rel = jnp.where(idx8 < n_eps, rel, _NEG)
    mx = jnp.max(rel, axis=0, keepdims=True)
    bidx = jnp.min(jnp.where(rel == mx, idx8, jnp.int32(2**30)),
                   axis=0, keepdims=True)
    oh8 = (idx8 == bidx).astype(jnp.float32)
    ep_content = jnp.sum(eps * oh8, axis=0, keepdims=True)
    ep_td = jnp.sum(td * oh8, axis=0, keepdims=True)

    wg1 = wg1_ref[...]
    x1 = jnp.abs(tde_ref[...])
    g = jnp.tanh(wg1[:, 0:1] * best_val[0] + wg1[:, 1:2] * x1
                 + wg1[:, 2:3] * ep_td + bg1_ref[...])
    alpha = jnp.tanh(jnp.sum(wg2_ref[...] * g, axis=0, keepdims=True)
                     + bg2_ref[...])

    delta = jnp.sum(wrbuf[...] * ep_content, axis=1, keepdims=True) \
        + br_ref[...]
    newpfc_ref[...] = pfc_col_ref[...] + alpha * delta
    alpha_ref[...] = alpha

    ii = jax.lax.broadcasted_iota(jnp.int32, (N_SLOTS, 1), 0)
    onehot_ref[...] = (ii == slot).astype(jnp.float32)

    nm = jnp.sum(wn_ref[...] * ep_content, axis=1, keepdims=True) + bn_ref[...]
    rows = jax.lax.broadcasted_iota(jnp.int32, (3 * N_PATCHES, 1), 0)
    hi = jnp.where(rows < 2 * N_PATCHES, 1.0, 0.5)
    nm_ref[...] = jnp.clip(nm, 0.1, hi)


def kernel(activation_summary, pfc_state, current_td_error, prototypes,
           log_temperature, W0, b0, W2, b2, episodes, ep_td_errors, ep_count,
           Wg1, bg1, Wg2, bg2, Wr, br, Wn, bn):
    f32 = jnp.float32
    combined = jnp.concatenate(
        [activation_summary.reshape(1, KEY_DIM), pfc_state], axis=1)

    vm = pl.BlockSpec(memory_space=pl.ANY)
    newpfc, alpha11, onehot, nm = pl.pallas_call(
        _fused_kernel,
        in_specs=[pl.BlockSpec(memory_space=pltpu.VMEM)] * 13
        + [vm] * 7,
        out_specs=[pl.BlockSpec(memory_space=pltpu.VMEM)] * 4,
        out_shape=[
            jax.ShapeDtypeStruct((PFC_DIM, 1), f32),
            jax.ShapeDtypeStruct((1, 1), f32),
            jax.ShapeDtypeStruct((N_SLOTS, 1), f32),
            jax.ShapeDtypeStruct((3 * N_PATCHES, 1), f32),
        ],
        scratch_shapes=[
            pltpu.VMEM((NRING, BM, H_DIM), f32),
            pltpu.VMEM((PFC_DIM, D_MEM), f32),
            pltpu.VMEM((1, EPS, D_MEM), f32),
            pltpu.VMEM((1, EPS, 1), f32),
            pltpu.VMEM((1, 1, 1), jnp.int32),
            pltpu.VMEM((1, H_DIM), f32),
            pltpu.VMEM((1, KEY_DIM), f32),
            pltpu.SMEM((1,), f32),
            pltpu.SMEM((1,), jnp.int32),
            pltpu.SemaphoreType.DMA((NRING,)),
            pltpu.SemaphoreType.DMA,
        ],
    )(combined, b0.reshape(1, H_DIM), b2.reshape(1, KEY_DIM),
      pfc_state, pfc_state.reshape(PFC_DIM, 1),
      current_td_error.reshape(1, 1), Wg1, bg1.reshape(16, 1),
      Wg2.reshape(16, 1), bg2.reshape(1, 1), br.reshape(PFC_DIM, 1),
      Wn, bn.reshape(3 * N_PATCHES, 1),
      W0, W2, prototypes, episodes, ep_td_errors.reshape(N_SLOTS, EPS, 1),
      ep_count.reshape(N_SLOTS, 1, 1), Wr)

    new_pfc = newpfc.reshape(1, PFC_DIM)
    alpha = alpha11.reshape(())
    one_hot_st = onehot.reshape(N_SLOTS)
    nmflat = nm.reshape(3 * N_PATCHES)
    eta = nmflat[0:N_PATCHES]
    decay = nmflat[N_PATCHES:2 * N_PATCHES]
    expl = nmflat[2 * N_PATCHES:]
    return (new_pfc, alpha, one_hot_st, eta, decay, expl)
